# baseline (speedup 1.0000x reference)
import functools

import jax
import jax.numpy as jnp
from jax import lax
from jax.experimental import pallas as pl
from jax.experimental.pallas import tpu as pltpu
from jax.experimental.pallas import tpu_sc as plsc

_NUM_ROWS = 100000
_NUM_UPD = 16384
_D = 128
_NC = 2
_NS = 16
_NW = _NC * _NS
_L = 16
_UPW = _NUM_UPD // _NW
_CH = 128
_NCH = _UPW // _CH
_G = 8
_NB = _NUM_UPD // _L


def _wid():
    return lax.axis_index("s") * _NC + lax.axis_index("c")


_CPB = 25000


def _copy_body(x_ref, o_ref):
    o_ref[...] = x_ref[...]


@functools.cache
def _tc_copy():
    return pl.pallas_call(
        _copy_body,
        grid=(_NUM_ROWS // _CPB,),
        in_specs=[pl.BlockSpec((_CPB, _D), lambda i: (i, 0))],
        out_specs=pl.BlockSpec((_CPB, _D), lambda i: (i, 0)),
        out_shape=jax.ShapeDtypeStruct((_NUM_ROWS, _D), jnp.float32),
    )


def _winner_body(idx_hbm, w_hbm, idx_v, mark_v, w_v, sem):
    pltpu.sync_copy(idx_hbm, idx_v)
    lanes = lax.iota(jnp.int32, _L)

    @pl.loop(0, _NB // _G)
    def _group(g):
        b0 = g * _G
        vs, poss = [], []
        for j in range(_G):
            v = idx_v[pl.ds((b0 + j) * _L, _L)]
            pos = (b0 + j) * _L + lanes
            plsc.store_scatter(mark_v, [v], pos)
            vs.append(v)
            poss.append(pos)
        pend = jnp.int32(0)
        for j in range(_G):
            got = plsc.load_gather(mark_v, [vs[j]])
            pend = jnp.maximum(pend, jnp.max(poss[j] - got))

        @pl.when(pend > 0)
        def _fixup():
            @pl.loop(b0, b0 + _G)
            def _fix(b):
                v = idx_v[pl.ds(b * _L, _L)]
                pos = b * _L + lanes

                def _pending(gt):
                    return jnp.max(pos - gt) > 0

                def _improve(gt):
                    plsc.store_scatter(mark_v, [v], pos, mask=pos > gt)
                    return plsc.load_gather(mark_v, [v])

                lax.while_loop(_pending, _improve,
                               plsc.load_gather(mark_v, [v]))

    base = _wid() * _UPW
    for k in range(_UPW // _L):
        v = idx_v[pl.ds(base + k * _L, _L)]
        w_v[k * _L:(k + 1) * _L] = plsc.load_gather(mark_v, [v])
    pltpu.sync_copy(w_v, w_hbm.at[pl.ds(base, _UPW)])


def _scatter_body(out_hbm, idx_hbm, w_hbm, upd_hbm,
                  stage_v, wflat, sidx, rows, gsems, ssem):
    base = _wid() * _UPW
    pltpu.sync_copy(w_hbm.at[pl.ds(base, _UPW)], wflat)
    pltpu.sync_copy(idx_hbm.at[pl.ds(base, _UPW)], stage_v)
    for c in range(_NCH):
        for k in range(_CH // _L):
            sidx[c, k * _L:(k + 1) * _L] = stage_v[pl.ds(c * _CH + k * _L, _L)]

    gathers = [
        pltpu.async_copy(upd_hbm.at[wflat.at[pl.ds(c * _CH, _CH)]],
                         rows[c], gsems[c])
        for c in range(_NCH)
    ]
    scatters = []
    for c in range(_NCH):
        gathers[c].wait()
        scatters.append(
            pltpu.async_copy(rows[c], out_hbm.at[sidx.at[c]], ssem))
    for s in scatters:
        s.wait()


def _nop_body(o_hbm, v, sem):
    @pl.when(_wid() == 0)
    def _():
        v[...] = jnp.zeros((_L,), jnp.int32)
        pltpu.sync_copy(v, o_hbm)


@functools.cache
def _kernels():
    mesh = plsc.VectorSubcoreMesh(
        core_axis_name="c", subcore_axis_name="s",
        num_cores=_NC, num_subcores=_NS,
    )
    params = pltpu.CompilerParams(needs_layout_passes=False)
    nop = pl.kernel(
        _nop_body,
        out_type=jax.ShapeDtypeStruct((_L,), jnp.int32),
        mesh=mesh,
        compiler_params=params,
        scratch_types=[
            pltpu.VMEM((_L,), jnp.int32),
            pltpu.SemaphoreType.DMA,
        ],
    )
    winner = pl.kernel(
        _winner_body,
        out_type=jax.ShapeDtypeStruct((_NUM_UPD,), jnp.int32),
        mesh=mesh,
        compiler_params=params,
        scratch_types=[
            pltpu.VMEM((_NUM_UPD,), jnp.int32),
            pltpu.VMEM((_NUM_ROWS,), jnp.int32),
            pltpu.VMEM((_UPW,), jnp.int32),
            pltpu.SemaphoreType.DMA,
        ],
    )
    scatter = pl.kernel(
        _scatter_body,
        out_type=(),
        mesh=mesh,
        compiler_params=params,
        scratch_types=[
            pltpu.VMEM((_UPW,), jnp.int32),
            pltpu.VMEM((_UPW,), jnp.int32),
            pltpu.VMEM((_NCH, _CH), jnp.int32),
            [pltpu.VMEM((_CH, _D), jnp.float32)] * _NCH,
            [pltpu.SemaphoreType.DMA] * _NCH,
            pltpu.SemaphoreType.DMA,
        ],
    )
    return nop, winner, scatter


def kernel(data, indices, updates):
    idx = indices.reshape(-1).astype(jnp.int32)
    nop, winner, scatter = _kernels()
    z = nop()
    idx = idx + z[0]
    w = winner(idx)
    out_ref = jax.new_ref(_tc_copy()(data))
    scatter(out_ref, idx, w, updates)
    return out_ref[...]


# speedup vs baseline: 1.0789x; 1.0789x over previous
import functools

import jax
import jax.numpy as jnp
from jax import lax
from jax.experimental import pallas as pl
from jax.experimental.pallas import tpu as pltpu
from jax.experimental.pallas import tpu_sc as plsc

_NUM_ROWS = 100000
_NUM_UPD = 16384
_D = 128
_NC = 2
_NS = 16
_NW = _NC * _NS
_L = 16
_UPW = _NUM_UPD // _NW
_CH = 128
_NCH = _UPW // _CH
_G = 8
_NB = _NUM_UPD // _L


def _wid():
    return lax.axis_index("s") * _NC + lax.axis_index("c")


_CPB = 25000


def _copy_body(x_ref, o_ref):
    o_ref[...] = x_ref[...]


@functools.cache
def _tc_copy():
    return pl.pallas_call(
        _copy_body,
        grid=(_NUM_ROWS // _CPB,),
        in_specs=[pl.BlockSpec((_CPB, _D), lambda i: (i, 0))],
        out_specs=pl.BlockSpec((_CPB, _D), lambda i: (i, 0)),
        out_shape=jax.ShapeDtypeStruct((_NUM_ROWS, _D), jnp.float32),
    )


def _winner_body(idx_hbm, w_hbm, idx_v, mark_v, w_v, sem):
    pltpu.sync_copy(idx_hbm, idx_v)
    lanes = lax.iota(jnp.int32, _L)

    @pl.loop(0, _NB // _G)
    def _group(g):
        b0 = g * _G
        vs, poss = [], []
        for j in range(_G):
            v = idx_v[pl.ds((b0 + j) * _L, _L)]
            pos = (b0 + j) * _L + lanes
            plsc.store_scatter(mark_v, [v], pos)
            vs.append(v)
            poss.append(pos)
        pend = jnp.int32(0)
        for j in range(_G):
            got = plsc.load_gather(mark_v, [vs[j]])
            pend = jnp.maximum(pend, jnp.max(poss[j] - got))

        @pl.when(pend > 0)
        def _fixup():
            @pl.loop(b0, b0 + _G)
            def _fix(b):
                v = idx_v[pl.ds(b * _L, _L)]
                pos = b * _L + lanes

                def _pending(gt):
                    return jnp.max(pos - gt) > 0

                def _improve(gt):
                    plsc.store_scatter(mark_v, [v], pos, mask=pos > gt)
                    return plsc.load_gather(mark_v, [v])

                lax.while_loop(_pending, _improve,
                               plsc.load_gather(mark_v, [v]))

    base = _wid() * _UPW
    for k in range(_UPW // _L):
        v = idx_v[pl.ds(base + k * _L, _L)]
        w_v[k * _L:(k + 1) * _L] = plsc.load_gather(mark_v, [v])
    pltpu.sync_copy(w_v, w_hbm.at[pl.ds(base, _UPW)])


def _scatter_body(out_hbm, idx_hbm, w_hbm, upd_hbm,
                  stage_v, wflat, sidx, rows, gsems, ssem):
    base = _wid() * _UPW
    pltpu.sync_copy(w_hbm.at[pl.ds(base, _UPW)], wflat)
    pltpu.sync_copy(idx_hbm.at[pl.ds(base, _UPW)], stage_v)
    for c in range(_NCH):
        for k in range(_CH // _L):
            sidx[c, k * _L:(k + 1) * _L] = stage_v[pl.ds(c * _CH + k * _L, _L)]

    gathers = [
        pltpu.async_copy(upd_hbm.at[wflat.at[pl.ds(c * _CH, _CH)]],
                         rows[c], gsems[c])
        for c in range(_NCH)
    ]
    scatters = []
    for c in range(_NCH):
        gathers[c].wait()
        scatters.append(
            pltpu.async_copy(rows[c], out_hbm.at[sidx.at[c]], ssem))
    for s in scatters:
        s.wait()


@functools.cache
def _kernels():
    mesh = plsc.VectorSubcoreMesh(
        core_axis_name="c", subcore_axis_name="s",
        num_cores=_NC, num_subcores=_NS,
    )
    params = pltpu.CompilerParams(needs_layout_passes=False)
    winner = pl.kernel(
        _winner_body,
        out_type=jax.ShapeDtypeStruct((_NUM_UPD,), jnp.int32),
        mesh=mesh,
        compiler_params=params,
        scratch_types=[
            pltpu.VMEM((_NUM_UPD,), jnp.int32),
            pltpu.VMEM((_NUM_ROWS,), jnp.int32),
            pltpu.VMEM((_UPW,), jnp.int32),
            pltpu.SemaphoreType.DMA,
        ],
    )
    scatter = pl.kernel(
        _scatter_body,
        out_type=(),
        mesh=mesh,
        compiler_params=params,
        scratch_types=[
            pltpu.VMEM((_UPW,), jnp.int32),
            pltpu.VMEM((_UPW,), jnp.int32),
            pltpu.VMEM((_NCH, _CH), jnp.int32),
            [pltpu.VMEM((_CH, _D), jnp.float32)] * _NCH,
            [pltpu.SemaphoreType.DMA] * _NCH,
            pltpu.SemaphoreType.DMA,
        ],
    )
    return winner, scatter


def kernel(data, indices, updates):
    idx = indices.reshape(-1).astype(jnp.int32)
    winner, scatter = _kernels()
    w = winner(idx)
    out_ref = jax.new_ref(_tc_copy()(data))
    scatter(out_ref, idx, w, updates)
    return jax.freeze(out_ref)


# speedup vs baseline: 1.0831x; 1.0039x over previous
import functools

import jax
import jax.numpy as jnp
from jax import lax
from jax.experimental import pallas as pl
from jax.experimental.pallas import tpu as pltpu
from jax.experimental.pallas import tpu_sc as plsc

_NUM_ROWS = 100000
_NUM_UPD = 16384
_D = 128
_NC = 2
_NS = 16
_NW = _NC * _NS
_L = 16
_UPW = _NUM_UPD // _NW
_CH = 128
_NCH = _UPW // _CH
_G = 8
_NB = _NUM_UPD // _L


def _wid():
    return lax.axis_index("s") * _NC + lax.axis_index("c")


_CPB = 25000


def _copy_body(x_ref, o_ref):
    o_ref[...] = x_ref[...]


@functools.cache
def _tc_copy():
    return pl.pallas_call(
        _copy_body,
        grid=(_NUM_ROWS // _CPB,),
        in_specs=[pl.BlockSpec((_CPB, _D), lambda i: (i, 0))],
        out_specs=pl.BlockSpec((_CPB, _D), lambda i: (i, 0)),
        out_shape=jax.ShapeDtypeStruct((_NUM_ROWS, _D), jnp.float32),
    )


def _winner_body(idx_hbm, w_hbm, idx_v, mark_v, w_v, sem):
    pltpu.sync_copy(idx_hbm, idx_v)
    lanes = lax.iota(jnp.int32, _L)

    @pl.loop(0, _NB // _G)
    def _group(g):
        b0 = g * _G
        vs, poss = [], []
        for j in range(_G):
            v = idx_v[pl.ds((b0 + j) * _L, _L)]
            pos = (b0 + j) * _L + lanes
            plsc.store_scatter(mark_v, [v], pos)
            vs.append(v)
            poss.append(pos)
        pend = jnp.int32(0)
        for j in range(_G):
            got = plsc.load_gather(mark_v, [vs[j]])
            pend = jnp.maximum(pend, jnp.max(poss[j] - got))

        @pl.when(pend > 0)
        def _fixup():
            @pl.loop(b0, b0 + _G)
            def _fix(b):
                v = idx_v[pl.ds(b * _L, _L)]
                pos = b * _L + lanes

                def _pending(gt):
                    return jnp.max(pos - gt) > 0

                def _improve(gt):
                    plsc.store_scatter(mark_v, [v], pos, mask=pos > gt)
                    return plsc.load_gather(mark_v, [v])

                lax.while_loop(_pending, _improve,
                               plsc.load_gather(mark_v, [v]))

    base = _wid() * _UPW
    for k in range(_UPW // _L):
        v = idx_v[pl.ds(base + k * _L, _L)]
        w_v[k * _L:(k + 1) * _L] = plsc.load_gather(mark_v, [v])
    pltpu.sync_copy(w_v, w_hbm.at[pl.ds(base, _UPW)])


def _scatter_body(out_hbm, idx_hbm, w_hbm, upd_hbm,
                  stage_v, wflat, sidx, rows, gsems, ssem):
    base = _wid() * _UPW
    pltpu.sync_copy(w_hbm.at[pl.ds(base, _UPW)], wflat)
    gathers = [
        pltpu.async_copy(upd_hbm.at[wflat.at[pl.ds(c * _CH, _CH)]],
                         rows[c], gsems[c])
        for c in range(_NCH)
    ]
    pltpu.sync_copy(idx_hbm.at[pl.ds(base, _UPW)], stage_v)
    for c in range(_NCH):
        for k in range(_CH // _L):
            sidx[c, k * _L:(k + 1) * _L] = stage_v[pl.ds(c * _CH + k * _L, _L)]
    scatters = []
    for c in range(_NCH):
        gathers[c].wait()
        scatters.append(
            pltpu.async_copy(rows[c], out_hbm.at[sidx.at[c]], ssem))
    for s in scatters:
        s.wait()


@functools.cache
def _kernels():
    mesh = plsc.VectorSubcoreMesh(
        core_axis_name="c", subcore_axis_name="s",
        num_cores=_NC, num_subcores=_NS,
    )
    params = pltpu.CompilerParams(needs_layout_passes=False)
    winner = pl.kernel(
        _winner_body,
        out_type=jax.ShapeDtypeStruct((_NUM_UPD,), jnp.int32),
        mesh=mesh,
        compiler_params=params,
        scratch_types=[
            pltpu.VMEM((_NUM_UPD,), jnp.int32),
            pltpu.VMEM((_NUM_ROWS,), jnp.int32),
            pltpu.VMEM((_UPW,), jnp.int32),
            pltpu.SemaphoreType.DMA,
        ],
    )
    scatter = pl.kernel(
        _scatter_body,
        out_type=(),
        mesh=mesh,
        compiler_params=params,
        scratch_types=[
            pltpu.VMEM((_UPW,), jnp.int32),
            pltpu.VMEM((_UPW,), jnp.int32),
            pltpu.VMEM((_NCH, _CH), jnp.int32),
            [pltpu.VMEM((_CH, _D), jnp.float32)] * _NCH,
            [pltpu.SemaphoreType.DMA] * _NCH,
            pltpu.SemaphoreType.DMA,
        ],
    )
    return winner, scatter


def kernel(data, indices, updates):
    idx = indices.reshape(-1).astype(jnp.int32)
    winner, scatter = _kernels()
    w = winner(idx)
    out_ref = jax.new_ref(_tc_copy()(data))
    scatter(out_ref, idx, w, updates)
    return jax.freeze(out_ref)
